# baseline (speedup 1.0000x reference)
import jax
import jax.numpy as jnp
from jax.experimental import pallas as pl

D = 768
P, Q, S = 7, 1, 69
N = 5


def _concat_body(p_ref, q_ref, s_ref, o_ref):
    i = pl.program_id(0)
    o_ref[:, :P, :] = p_ref[...]
    o_ref[:, P : P + Q, :] = q_ref[i, :][None, None, :]
    o_ref[:, P + Q :, :] = s_ref[...]


def kernel(embedding_prefix, learnable_quality, embedding_suffix):
    return pl.pallas_call(
        _concat_body,
        grid=(N,),
        in_specs=[
            pl.BlockSpec((1, P, D), lambda i: (i, 0, 0)),
            pl.BlockSpec((N, D), lambda i: (0, 0)),
            pl.BlockSpec((1, S, D), lambda i: (i, 0, 0)),
        ],
        out_specs=pl.BlockSpec((1, P + Q + S, D), lambda i: (i, 0, 0)),
        out_shape=jax.ShapeDtypeStruct((N, P + Q + S, D), jnp.float32),
    )(embedding_prefix, learnable_quality, embedding_suffix)


# speedup vs baseline: 1.6404x; 1.6404x over previous
import jax
import jax.numpy as jnp
from jax.experimental import pallas as pl

D = 768
P, Q, S = 7, 1, 69
N = 5


def _concat_body(p_ref, q_ref, s_ref, o_ref):
    o_ref[:, :P, :] = p_ref[...]
    o_ref[:, P : P + Q, :] = q_ref[...][:, None, :]
    o_ref[:, P + Q :, :] = s_ref[...]


def kernel(embedding_prefix, learnable_quality, embedding_suffix):
    return pl.pallas_call(
        _concat_body,
        out_shape=jax.ShapeDtypeStruct((N, P + Q + S, D), jnp.float32),
    )(embedding_prefix, learnable_quality, embedding_suffix)


# speedup vs baseline: 1.6564x; 1.0098x over previous
import jax
import jax.numpy as jnp
from jax.experimental import pallas as pl
from jax.experimental.pallas import tpu as pltpu

D = 768
P, Q, S = 7, 1, 69
N = 5


def _concat_body(p_ref, q_ref, s_ref, o_ref, sem_p, sem_q, sem_s):
    cp = pltpu.make_async_copy(p_ref, o_ref.at[:, :P, :], sem_p)
    cq = pltpu.make_async_copy(q_ref, o_ref.at[:, P, :], sem_q)
    cs = pltpu.make_async_copy(s_ref, o_ref.at[:, P + Q :, :], sem_s)
    cp.start()
    cq.start()
    cs.start()
    cp.wait()
    cq.wait()
    cs.wait()


def kernel(embedding_prefix, learnable_quality, embedding_suffix):
    return pl.pallas_call(
        _concat_body,
        out_shape=jax.ShapeDtypeStruct((N, P + Q + S, D), jnp.float32),
        in_specs=[
            pl.BlockSpec(memory_space=pl.ANY),
            pl.BlockSpec(memory_space=pl.ANY),
            pl.BlockSpec(memory_space=pl.ANY),
        ],
        scratch_shapes=[
            pltpu.SemaphoreType.DMA,
            pltpu.SemaphoreType.DMA,
            pltpu.SemaphoreType.DMA,
        ],
    )(embedding_prefix, learnable_quality, embedding_suffix)


# speedup vs baseline: 1.7838x; 1.0769x over previous
import jax
import jax.numpy as jnp
from jax.experimental import pallas as pl
from jax.experimental.pallas import tpu as pltpu

D = 768
P, Q, S = 7, 1, 69
N = 5
T = P + Q + S
CHUNKS = ((0, 24), (24, 24), (48, 21))


def _concat_body(p_ref, q_ref, s_ref, o_ref, v_ref, sem_in, sem_out):
    ip = pltpu.make_async_copy(p_ref, v_ref.at[:, :P, :], sem_in.at[0])
    iq = pltpu.make_async_copy(q_ref, v_ref.at[:, P, :], sem_in.at[1])
    i_s = [
        pltpu.make_async_copy(
            s_ref.at[:, off : off + sz, :],
            v_ref.at[:, P + Q + off : P + Q + off + sz, :],
            sem_in.at[2 + k],
        )
        for k, (off, sz) in enumerate(CHUNKS)
    ]
    ip.start()
    iq.start()
    for c in i_s:
        c.start()

    ip.wait()
    iq.wait()
    o0 = pltpu.make_async_copy(
        v_ref.at[:, : P + Q, :], o_ref.at[:, : P + Q, :], sem_out.at[0]
    )
    o0.start()
    outs = [o0]
    for k, (off, sz) in enumerate(CHUNKS):
        i_s[k].wait()
        ok = pltpu.make_async_copy(
            v_ref.at[:, P + Q + off : P + Q + off + sz, :],
            o_ref.at[:, P + Q + off : P + Q + off + sz, :],
            sem_out.at[1 + k],
        )
        ok.start()
        outs.append(ok)
    for c in outs:
        c.wait()


def kernel(embedding_prefix, learnable_quality, embedding_suffix):
    return pl.pallas_call(
        _concat_body,
        out_shape=jax.ShapeDtypeStruct((N, T, D), jnp.float32),
        in_specs=[
            pl.BlockSpec(memory_space=pl.ANY),
            pl.BlockSpec(memory_space=pl.ANY),
            pl.BlockSpec(memory_space=pl.ANY),
        ],
        out_specs=pl.BlockSpec(memory_space=pl.ANY),
        scratch_shapes=[
            pltpu.VMEM((N, T, D), jnp.float32),
            pltpu.SemaphoreType.DMA((5,)),
            pltpu.SemaphoreType.DMA((4,)),
        ],
    )(embedding_prefix, learnable_quality, embedding_suffix)
